# baseline (speedup 1.0000x reference)
import functools

import jax
import jax.numpy as jnp
from jax import lax
from jax.experimental import pallas as pl
from jax.experimental.pallas import tpu as pltpu
from jax.experimental.pallas import tpu_sc as plsc

NBATCH = 4096
SEQ = 50
D = 128

_info = plsc.get_sparse_core_info()
NC, NS = _info.num_cores, _info.num_subcores
NW = NC * NS
BAT_PER_W = NBATCH // NW
NBUF = 2
N_GROUPS = SEQ // NBUF

_mesh = plsc.VectorSubcoreMesh(core_axis_name="c", subcore_axis_name="s")


@functools.partial(
    pl.kernel,
    mesh=_mesh,
    out_type=jax.ShapeDtypeStruct((SEQ, NBATCH, D), jnp.float32),
    scratch_types=[
        pltpu.VMEM((SEQ, BAT_PER_W), jnp.int32),
        pltpu.VMEM((NBUF, BAT_PER_W, D), jnp.float32),
    ]
    + [pltpu.SemaphoreType.DMA] * (2 * NBUF),
)
def _gather_kernel(idx_hbm, table_hbm, out_hbm, idx_v, rows_v, *sems):
    gsem = sems[:NBUF]
    osem = sems[NBUF:]
    wid = lax.axis_index("s") * NC + lax.axis_index("c")
    bat0 = wid * BAT_PER_W

    pltpu.sync_copy(idx_hbm.at[:, pl.ds(bat0, BAT_PER_W)], idx_v)

    def gather_desc(s, b):
        return pltpu.make_async_copy(
            table_hbm.at[idx_v.at[s]], rows_v.at[b], gsem[b]
        )

    def out_desc(s, b):
        return pltpu.make_async_copy(
            rows_v.at[b], out_hbm.at[s, pl.ds(bat0, BAT_PER_W)], osem[b]
        )

    for b in range(NBUF):
        gather_desc(b, b).start()

    def body(g, _):
        for b in range(NBUF):
            s = g * NBUF + b
            gather_desc(s, b).wait()
            out_desc(s, b).start()
        for b in range(NBUF):
            s = g * NBUF + b
            out_desc(s, b).wait()

            @pl.when(g + 1 < N_GROUPS)
            def _():
                gather_desc(s + NBUF, b).start()

        return 0

    lax.fori_loop(0, N_GROUPS, body, 0)


def kernel(X, W):
    idxT = X.T.astype(jnp.int32)
    outT = _gather_kernel(idxT, W)
    return outT.transpose(1, 0, 2)


# speedup vs baseline: 1.1060x; 1.1060x over previous
import functools

import jax
import jax.numpy as jnp
from jax import lax
from jax.experimental import pallas as pl
from jax.experimental.pallas import tpu as pltpu
from jax.experimental.pallas import tpu_sc as plsc

NBATCH = 4096
SEQ = 50
D = 128

_info = plsc.get_sparse_core_info()
NC, NS = _info.num_cores, _info.num_subcores
NW = NC * NS
BAT_PER_W = NBATCH // NW
SPLIT = 2
CB = BAT_PER_W // SPLIT
N_CHUNKS = SEQ * SPLIT
NBUF = 10
N_GROUPS = N_CHUNKS // NBUF

_mesh = plsc.VectorSubcoreMesh(core_axis_name="c", subcore_axis_name="s")


@functools.partial(
    pl.kernel,
    mesh=_mesh,
    out_type=jax.ShapeDtypeStruct((SEQ, NBATCH, D), jnp.float32),
    scratch_types=[
        pltpu.VMEM((SEQ, BAT_PER_W), jnp.int32),
        pltpu.VMEM((NBUF, CB, D), jnp.float32),
    ]
    + [pltpu.SemaphoreType.DMA] * (2 * NBUF),
)
def _gather_kernel(idx_hbm, table_hbm, out_hbm, idx_v, rows_v, *sems):
    gsem = sems[:NBUF]
    osem = sems[NBUF:]
    wid = lax.axis_index("s") * NC + lax.axis_index("c")
    bat0 = wid * BAT_PER_W

    pltpu.sync_copy(idx_hbm.at[:, pl.ds(bat0, BAT_PER_W)], idx_v)

    def gather_desc(c, b):
        s = c // SPLIT
        half = c % SPLIT
        return pltpu.make_async_copy(
            table_hbm.at[idx_v.at[s, pl.ds(half * CB, CB)]],
            rows_v.at[b],
            gsem[b],
        )

    def out_desc(c, b):
        s = c // SPLIT
        half = c % SPLIT
        return pltpu.make_async_copy(
            rows_v.at[b],
            out_hbm.at[s, pl.ds(bat0 + half * CB, CB)],
            osem[b],
        )

    for b in range(NBUF):
        gather_desc(b, b).start()

    def body(g, _):
        for b in range(NBUF):
            c = g * NBUF + b
            gather_desc(c, b).wait()
            out_desc(c, b).start()
        for b in range(NBUF):
            c = g * NBUF + b
            out_desc(c, b).wait()

            @pl.when(g + 1 < N_GROUPS)
            def _():
                gather_desc(c + NBUF, b).start()

        return 0

    lax.fori_loop(0, N_GROUPS, body, 0)


def kernel(X, W):
    idxT = X.T.astype(jnp.int32)
    outT = _gather_kernel(idxT, W)
    return outT.transpose(1, 0, 2)
